# baseline (speedup 1.0000x reference)
import functools

import jax
import jax.numpy as jnp
from jax import lax
from jax.experimental import pallas as pl
from jax.experimental.pallas import tpu as pltpu
from jax.experimental.pallas import tpu_sc as plsc

NUM_CORES = 2
NUM_SUBCORES = 16
NUM_WORKERS = NUM_CORES * NUM_SUBCORES
NBUF = 8
SPD = 2
LANES = 16


def _make_kernel(B, L, D):
    assert B % (NUM_WORKERS * SPD) == 0
    s_per_w = B // NUM_WORKERS
    d_per_w = s_per_w // SPD
    assert d_per_w % NBUF == 0
    rows_per_d = SPD * L
    inv_l = jnp.float32(1.0 / L)
    n_half = D // LANES
    out_rows_w = s_per_w * D // (4 * D)

    mesh = plsc.VectorSubcoreMesh(core_axis_name="c", subcore_axis_name="s")

    @functools.partial(
        pl.kernel,
        mesh=mesh,
        out_type=jax.ShapeDtypeStruct((B * D // (4 * D), 4 * D), jnp.float32),
        scratch_types=[
            pltpu.VMEM((d_per_w, rows_per_d), jnp.int32),
            pltpu.VMEM((NBUF, rows_per_d, D), jnp.bfloat16),
            pltpu.VMEM((out_rows_w, 4 * D), jnp.float32),
        ]
        + [pltpu.SemaphoreType.DMA] * NBUF,
        compiler_params=pltpu.CompilerParams(
            use_tc_tiling_on_sc=False, needs_layout_passes=False
        ),
    )
    def run(table_hbm, idx_hbm, out_hbm, idx_v, ring_v, out_v, *sems):
        wid = lax.axis_index("s") * NUM_CORES + lax.axis_index("c")
        base = wid * d_per_w

        pltpu.sync_copy(idx_hbm.at[pl.ds(base, d_per_w)], idx_v)

        def gather(d, b):
            return pltpu.make_async_copy(
                table_hbm.at[idx_v.at[d]], ring_v.at[b], sems[b]
            )

        for b in range(NBUF):
            gather(b, b).start()

        def reduce_rows(rows, base_t):
            accs = [
                [rows[base_t, pl.ds(h * LANES, LANES)] for h in range(n_half)],
                [rows[base_t + 1, pl.ds(h * LANES, LANES)] for h in range(n_half)],
            ]
            for t in range(2, L):
                c = accs[t % 2]
                for h in range(n_half):
                    c[h] += rows[base_t + t, pl.ds(h * LANES, LANES)]
            return [(accs[0][h] + accs[1][h]) * inv_l for h in range(n_half)]

        def outer(g, _):
            for b in range(NBUF):
                d = g * NBUF + b
                gather(d, b).wait()
                for sp in range(SPD):
                    pooled = reduce_rows(ring_v.at[b], sp * L)
                    s = d * SPD + sp
                    for h in range(n_half):
                        off = s * D + h * LANES
                        out_v[off // (4 * D), pl.ds(off % (4 * D), LANES)] = (
                            pooled[h]
                        )

                @pl.when(d + NBUF < d_per_w)
                def _():
                    gather(d + NBUF, b).start()

            return _

        lax.fori_loop(0, d_per_w // NBUF, outer, None)
        pltpu.sync_copy(out_v, out_hbm.at[pl.ds(wid * out_rows_w, out_rows_w)])

    return run


def kernel(indices, table):
    B, L = indices.shape
    V, D = table.shape
    idx = indices.astype(jnp.int32).reshape(B // SPD, SPD * L)
    out4 = _make_kernel(B, L, D)(table.astype(jnp.bfloat16), idx)
    return out4.reshape(B, D)


# speedup vs baseline: 1.0921x; 1.0921x over previous
import functools

import jax
import jax.numpy as jnp
from jax import lax
from jax.experimental import pallas as pl
from jax.experimental.pallas import tpu as pltpu
from jax.experimental.pallas import tpu_sc as plsc

NUM_CORES = 2
NUM_SUBCORES = 16
NUM_WORKERS = NUM_CORES * NUM_SUBCORES
NBUF = 8
SPD = 2
LANES = 16


def _make_kernel(B, L, D):
    assert B % (NUM_WORKERS * SPD) == 0
    s_per_w = B // NUM_WORKERS
    d_per_w = s_per_w // SPD
    assert d_per_w % NBUF == 0
    rows_per_d = SPD * L
    inv_l = jnp.float32(1.0 / L)
    n_half = D // LANES
    out_rows_w = s_per_w * D // (4 * D)

    mesh = plsc.VectorSubcoreMesh(core_axis_name="c", subcore_axis_name="s")

    @functools.partial(
        pl.kernel,
        mesh=mesh,
        out_type=jax.ShapeDtypeStruct((B * D // (4 * D), 4 * D), jnp.float32),
        scratch_types=[
            pltpu.VMEM((d_per_w, rows_per_d), jnp.int32),
            pltpu.VMEM((NBUF, rows_per_d, D), jnp.float32),
            pltpu.VMEM((out_rows_w, 4 * D), jnp.float32),
        ]
        + [pltpu.SemaphoreType.DMA] * NBUF,
        compiler_params=pltpu.CompilerParams(use_tc_tiling_on_sc=False),
    )
    def run(table_hbm, idx_hbm, out_hbm, idx_v, ring_v, out_v, *sems):
        wid = lax.axis_index("s") * NUM_CORES + lax.axis_index("c")
        base = wid * d_per_w

        pltpu.sync_copy(idx_hbm.at[pl.ds(base, d_per_w)], idx_v)

        def gather(d, b):
            return pltpu.make_async_copy(
                table_hbm.at[idx_v.at[d]], ring_v.at[b], sems[b]
            )

        for b in range(NBUF):
            gather(b, b).start()

        def reduce_rows(rows, base_t):
            accs = [
                [rows[base_t, pl.ds(h * LANES, LANES)] for h in range(n_half)],
                [rows[base_t + 1, pl.ds(h * LANES, LANES)] for h in range(n_half)],
            ]
            for t in range(2, L):
                c = accs[t % 2]
                for h in range(n_half):
                    c[h] += rows[base_t + t, pl.ds(h * LANES, LANES)]
            return [(accs[0][h] + accs[1][h]) * inv_l for h in range(n_half)]

        def outer(g, _):
            for b in range(NBUF):
                d = g * NBUF + b
                gather(d, b).wait()
                for sp in range(SPD):
                    pooled = reduce_rows(ring_v.at[b], sp * L)
                    s = d * SPD + sp
                    for h in range(n_half):
                        off = s * D + h * LANES
                        out_v[off // (4 * D), pl.ds(off % (4 * D), LANES)] = (
                            pooled[h]
                        )

                @pl.when(d + NBUF < d_per_w)
                def _():
                    gather(d + NBUF, b).start()

            return _

        lax.fori_loop(0, d_per_w // NBUF, outer, None)
        pltpu.sync_copy(out_v, out_hbm.at[pl.ds(wid * out_rows_w, out_rows_w)])

    return run


def kernel(indices, table):
    B, L = indices.shape
    V, D = table.shape
    idx = indices.astype(jnp.int32).reshape(B // SPD, SPD * L)
    out4 = _make_kernel(B, L, D)(table, idx)
    return out4.reshape(B, D)
